# Initial kernel scaffold; baseline (speedup 1.0000x reference)
import jax
import jax.numpy as jnp
from jax.experimental import pallas as pl


def kernel(x0, x1, edge_index0, edge_index1, t0, t1):
    raise NotImplementedError("write your pallas kernel here")


# speedup vs baseline: 6.0899x; 6.0899x over previous
import functools

import jax
import jax.numpy as jnp
from jax import lax
from jax.experimental import pallas as pl
from jax.experimental.pallas import tpu as pltpu
from jax.experimental.pallas import tpu_sc as plsc

N_NODES = 10000
N_EDGES = 320000
D_FEAT = 128

NC = 2
NS = 16
B = 128
CHUNK = 2 * N_EDGES // (NC * NS)
NBB = 32
NB = -(-CHUNK // (B * NBB)) * NBB
NG = NB // NBB
PAD_CHUNK = NB * B
N_PAD = 10240
ROWS_PER_TILE = N_PAD // NS
RQ = 128


def _sc_kernel(x_hbm, src_hbm, dst_hbm, t_hbm, out_hbm,
               src_v, dst_v, t_v, rows_v, acc, sem):
    c = lax.axis_index("c")
    s = lax.axis_index("s")
    w = c * NS + s
    row0 = s * ROWS_PER_TILE

    zero16 = jnp.zeros((16,), jnp.float32)

    def zrow(r, carry):
        for k in range(D_FEAT // 16):
            rows_v[r, pl.ds(k * 16, 16)] = zero16
        return carry

    lax.fori_loop(0, B, zrow, 0)
    for q in range(ROWS_PER_TILE // RQ):
        pltpu.sync_copy(rows_v, acc.at[pl.ds(row0 + q * RQ, RQ)])

    plsc.subcore_barrier()

    def group(g, gcarry):
        pltpu.sync_copy(src_hbm.at[w, pl.ds(g * NBB, NBB)], src_v)
        pltpu.sync_copy(dst_hbm.at[w, pl.ds(g * NBB, NBB)], dst_v)
        pltpu.sync_copy(t_hbm.at[w, pl.ds(g * NBB, NBB)], t_v)

        def batch(j, carry):
            pltpu.async_copy(x_hbm.at[src_v.at[j]], rows_v, sem).wait()

            def block(bb, inner):
                t16 = t_v[j, pl.ds(bb * 16, 16)]
                for l in range(16):
                    tl = t16[l]
                    e = bb * 16 + l
                    for k in range(D_FEAT // 16):
                        sl = pl.ds(k * 16, 16)
                        rows_v[e, sl] = rows_v[e, sl] * tl
                return inner

            lax.fori_loop(0, B // 16, block, 0)

            pltpu.sync_copy(rows_v, acc.at[dst_v.at[j]], add=True)
            return carry

        lax.fori_loop(0, NBB, batch, 0)
        return gcarry

    lax.fori_loop(0, NG, group, 0)
    plsc.subcore_barrier()

    for q in range(ROWS_PER_TILE // RQ):
        r = row0 + q * RQ
        pltpu.sync_copy(acc.at[pl.ds(r, RQ)], rows_v)
        pltpu.sync_copy(rows_v, out_hbm.at[c, pl.ds(r, RQ)])


@jax.jit
def kernel(x0, x1, edge_index0, edge_index1, t0, t1):
    x = jnp.concatenate([x0, x1], axis=0)
    src = jnp.concatenate([edge_index0[0].astype(jnp.int32),
                           edge_index1[0].astype(jnp.int32) + N_NODES])
    dst = jnp.concatenate([edge_index0[1].astype(jnp.int32),
                           edge_index1[1].astype(jnp.int32)])
    t = jnp.concatenate([t0[:, 0], t1[:, 0]])

    pad = NS * PAD_CHUNK - N_EDGES
    pad_src = (jnp.arange(pad, dtype=jnp.int32) * 37) % N_NODES
    pad_dst = (jnp.arange(pad, dtype=jnp.int32) * 53) % N_NODES
    pad_t = jnp.zeros((pad,), jnp.float32)
    src = jnp.concatenate([src[:N_EDGES], pad_src,
                           src[N_EDGES:], pad_src + N_NODES])
    dst = jnp.concatenate([dst[:N_EDGES], pad_dst,
                           dst[N_EDGES:], pad_dst])
    t = jnp.concatenate([t[:N_EDGES], pad_t, t[N_EDGES:], pad_t])

    src = src.reshape(NC * NS, NB, B)
    dst = dst.reshape(NC * NS, NB, B)
    t = t.reshape(NC * NS, NB, B)

    mesh = plsc.VectorSubcoreMesh(core_axis_name="c", subcore_axis_name="s")
    run = pl.kernel(
        _sc_kernel,
        out_type=jax.ShapeDtypeStruct((NC, N_PAD, D_FEAT), jnp.float32),
        mesh=mesh,
        scratch_types=[
            pltpu.VMEM((NBB, B), jnp.int32),
            pltpu.VMEM((NBB, B), jnp.int32),
            pltpu.VMEM((NBB, B), jnp.float32),
            pltpu.VMEM((B, D_FEAT), jnp.float32),
            pltpu.VMEM_SHARED((N_PAD, D_FEAT), jnp.float32),
            pltpu.SemaphoreType.DMA,
        ],
    )
    out = run(x, src, dst, t)
    return jnp.swapaxes(out[:, :N_NODES, :], 0, 1)


# speedup vs baseline: 8.7893x; 1.4433x over previous
import functools

import jax
import jax.numpy as jnp
from jax import lax
from jax.experimental import pallas as pl
from jax.experimental.pallas import tpu as pltpu
from jax.experimental.pallas import tpu_sc as plsc

N_NODES = 10000
N_EDGES = 320000
D_FEAT = 128

NC = 2
NS = 16
B = 128
CHUNK = 2 * N_EDGES // (NC * NS)
NBB = 16
NB = -(-CHUNK // (B * NBB)) * NBB
NG = NB // NBB
PAD_CHUNK = NB * B
N_PAD = 10240
ROWS_PER_TILE = N_PAD // NS
RQ = 128


def _sc_kernel(x_hbm, src_hbm, dst_hbm, t_hbm, out_hbm,
               src_v, dst_v, t_v, rows0, rows1, acc, sem0, sem1):
    rows_v = rows0
    c = lax.axis_index("c")
    s = lax.axis_index("s")
    w = c * NS + s
    row0 = s * ROWS_PER_TILE

    zero16 = jnp.zeros((16,), jnp.float32)

    def zrow(r, carry):
        for k in range(D_FEAT // 16):
            rows_v[r, pl.ds(k * 16, 16)] = zero16
        return carry

    lax.fori_loop(0, B, zrow, 0)
    for q in range(ROWS_PER_TILE // RQ):
        pltpu.sync_copy(rows_v, acc.at[pl.ds(row0 + q * RQ, RQ)])

    plsc.subcore_barrier()

    def scale_rows(rows, j):
        def block(bb, inner):
            t16 = t_v[j, pl.ds(bb * 16, 16)]
            for l in range(16):
                tl = t16[l]
                e = bb * 16 + l
                for k in range(D_FEAT // 16):
                    sl = pl.ds(k * 16, 16)
                    rows[e, sl] = rows[e, sl] * tl
            return inner

        lax.fori_loop(0, B // 16, block, 0)

    def group(g, gcarry):
        pltpu.sync_copy(src_hbm.at[w, pl.ds(g * NBB, NBB)], src_v)
        pltpu.sync_copy(dst_hbm.at[w, pl.ds(g * NBB, NBB)], dst_v)
        pltpu.sync_copy(t_hbm.at[w, pl.ds(g * NBB, NBB)], t_v)

        pltpu.async_copy(x_hbm.at[src_v.at[0]], rows0, sem0)

        def pair(p, carry):
            for b, (rows, sem, nrows, nsem) in enumerate(
                    ((rows0, sem0, rows1, sem1),
                     (rows1, sem1, rows0, sem0))):
                j = p * 2 + b
                pltpu.make_async_copy(x_hbm.at[src_v.at[j]], rows, sem).wait()

                @pl.when(j + 1 < NBB)
                def _():
                    pltpu.async_copy(x_hbm.at[src_v.at[j + 1]], nrows, nsem)

                scale_rows(rows, j)
                pltpu.sync_copy(rows, acc.at[dst_v.at[j]], add=True)
            return carry

        lax.fori_loop(0, NBB // 2, pair, 0)
        return gcarry

    lax.fori_loop(0, NG, group, 0)
    plsc.subcore_barrier()

    for q in range(ROWS_PER_TILE // RQ):
        r = row0 + q * RQ
        pltpu.sync_copy(acc.at[pl.ds(r, RQ)], rows_v)
        pltpu.sync_copy(rows_v, out_hbm.at[c, pl.ds(r, RQ)])


@jax.jit
def kernel(x0, x1, edge_index0, edge_index1, t0, t1):
    x = jnp.concatenate([x0, x1], axis=0)
    src = jnp.concatenate([edge_index0[0].astype(jnp.int32),
                           edge_index1[0].astype(jnp.int32) + N_NODES])
    dst = jnp.concatenate([edge_index0[1].astype(jnp.int32),
                           edge_index1[1].astype(jnp.int32)])
    t = jnp.concatenate([t0[:, 0], t1[:, 0]])

    pad = NS * PAD_CHUNK - N_EDGES
    pad_src = (jnp.arange(pad, dtype=jnp.int32) * 37) % N_NODES
    pad_dst = (jnp.arange(pad, dtype=jnp.int32) * 53) % N_NODES
    pad_t = jnp.zeros((pad,), jnp.float32)
    src = jnp.concatenate([src[:N_EDGES], pad_src,
                           src[N_EDGES:], pad_src + N_NODES])
    dst = jnp.concatenate([dst[:N_EDGES], pad_dst,
                           dst[N_EDGES:], pad_dst])
    t = jnp.concatenate([t[:N_EDGES], pad_t, t[N_EDGES:], pad_t])

    src = src.reshape(NC * NS, NB, B)
    dst = dst.reshape(NC * NS, NB, B)
    t = t.reshape(NC * NS, NB, B)

    mesh = plsc.VectorSubcoreMesh(core_axis_name="c", subcore_axis_name="s")
    run = pl.kernel(
        _sc_kernel,
        out_type=jax.ShapeDtypeStruct((NC, N_PAD, D_FEAT), jnp.float32),
        mesh=mesh,
        scratch_types=[
            pltpu.VMEM((NBB, B), jnp.int32),
            pltpu.VMEM((NBB, B), jnp.int32),
            pltpu.VMEM((NBB, B), jnp.float32),
            pltpu.VMEM((B, D_FEAT), jnp.float32),
            pltpu.VMEM((B, D_FEAT), jnp.float32),
            pltpu.VMEM_SHARED((N_PAD, D_FEAT), jnp.float32),
            pltpu.SemaphoreType.DMA,
            pltpu.SemaphoreType.DMA,
        ],
    )
    out = run(x, src, dst, t)
    return jnp.swapaxes(out[:, :N_NODES, :], 0, 1)


# speedup vs baseline: 8.9544x; 1.0188x over previous
import functools

import jax
import jax.numpy as jnp
from jax import lax
from jax.experimental import pallas as pl
from jax.experimental.pallas import tpu as pltpu
from jax.experimental.pallas import tpu_sc as plsc

N_NODES = 10000
N_EDGES = 320000
D_FEAT = 128

NC = 2
NS = 16
B = 112
CHUNK = 2 * N_EDGES // (NC * NS)
NBB = 12
NB = -(-CHUNK // (B * NBB)) * NBB
NG = NB // NBB
PAD_CHUNK = NB * B
N_PAD = 10240
ROWS_PER_TILE = N_PAD // NS
RQ = 80


def _sc_kernel(x_hbm, src_hbm, dst_hbm, t_hbm, out_hbm,
               src_v, dst_v, t_v, rows0, rows1, rows2, acc,
               gsem0, gsem1, gsem2, ssem0, ssem1, ssem2):
    c = lax.axis_index("c")
    s = lax.axis_index("s")
    w = c * NS + s
    row0 = s * ROWS_PER_TILE
    bufs = ((rows0, gsem0, ssem0), (rows1, gsem1, ssem1), (rows2, gsem2, ssem2))

    zero16 = jnp.zeros((16,), jnp.float32)

    def zrow(r, carry):
        for k in range(D_FEAT // 16):
            rows0[r, pl.ds(k * 16, 16)] = zero16
        return carry

    lax.fori_loop(0, RQ, zrow, 0)
    for q in range(ROWS_PER_TILE // RQ):
        pltpu.sync_copy(rows0.at[pl.ds(0, RQ)],
                        acc.at[pl.ds(row0 + q * RQ, RQ)])
    plsc.subcore_barrier()

    def scale_rows(rows, j):
        def block(bb, inner):
            t16 = t_v[j, pl.ds(bb * 16, 16)]
            for l in range(16):
                tl = t16[l]
                e = bb * 16 + l
                for k in range(D_FEAT // 16):
                    sl = pl.ds(k * 16, 16)
                    rows[e, sl] = rows[e, sl] * tl
            return inner

        lax.fori_loop(0, B // 16, block, 0)

    def group(g, gcarry):
        @pl.when(g > 0)
        def _():
            pltpu.make_async_copy(rows1, acc.at[dst_v.at[0]], ssem1).wait()
            pltpu.make_async_copy(rows2, acc.at[dst_v.at[0]], ssem2).wait()

        pltpu.sync_copy(src_hbm.at[w, g], src_v)
        pltpu.sync_copy(dst_hbm.at[w, g], dst_v)
        pltpu.sync_copy(t_hbm.at[w, g], t_v)

        pltpu.async_copy(x_hbm.at[src_v.at[0]], rows0, gsem0)

        def triple(p, carry):
            for b in range(3):
                jj = p * 3 + b
                rows, gsem, _ssem = bufs[b]
                nrows, ngsem, nssem = bufs[(b + 1) % 3]

                @pl.when(jj >= 2)
                def _():
                    pltpu.make_async_copy(
                        nrows, acc.at[dst_v.at[0]], nssem).wait()

                @pl.when(jj + 1 < NBB)
                def _():
                    pltpu.async_copy(x_hbm.at[src_v.at[jj + 1]], nrows, ngsem)

                pltpu.make_async_copy(
                    x_hbm.at[src_v.at[jj]], rows, gsem).wait()
                scale_rows(rows, jj)
                pltpu.async_copy(rows, acc.at[dst_v.at[jj]], _ssem, add=True)
            return carry

        lax.fori_loop(0, NBB // 3, triple, 0)
        return gcarry

    lax.fori_loop(0, NG, group, 0)

    pltpu.make_async_copy(rows1, acc.at[dst_v.at[0]], ssem1).wait()
    pltpu.make_async_copy(rows2, acc.at[dst_v.at[0]], ssem2).wait()
    plsc.subcore_barrier()

    for q in range(ROWS_PER_TILE // RQ):
        r = row0 + q * RQ
        pltpu.sync_copy(acc.at[pl.ds(r, RQ)], rows0.at[pl.ds(0, RQ)])
        pltpu.sync_copy(rows0.at[pl.ds(0, RQ)], out_hbm.at[c, pl.ds(r, RQ)])


@jax.jit
def kernel(x0, x1, edge_index0, edge_index1, t0, t1):
    x = jnp.concatenate([x0, x1], axis=0)
    src = jnp.concatenate([edge_index0[0].astype(jnp.int32),
                           edge_index1[0].astype(jnp.int32) + N_NODES])
    dst = jnp.concatenate([edge_index0[1].astype(jnp.int32),
                           edge_index1[1].astype(jnp.int32)])
    t = jnp.concatenate([t0[:, 0], t1[:, 0]])

    pad = NS * PAD_CHUNK - N_EDGES
    pad_src = (jnp.arange(pad, dtype=jnp.int32) * 37) % N_NODES
    pad_dst = (jnp.arange(pad, dtype=jnp.int32) * 53) % N_NODES
    pad_t = jnp.zeros((pad,), jnp.float32)
    src = jnp.concatenate([src[:N_EDGES], pad_src,
                           src[N_EDGES:], pad_src + N_NODES])
    dst = jnp.concatenate([dst[:N_EDGES], pad_dst,
                           dst[N_EDGES:], pad_dst])
    t = jnp.concatenate([t[:N_EDGES], pad_t, t[N_EDGES:], pad_t])

    src = src.reshape(NC * NS, NG, NBB, B)
    dst = dst.reshape(NC * NS, NG, NBB, B)
    t = t.reshape(NC * NS, NG, NBB, B)

    mesh = plsc.VectorSubcoreMesh(core_axis_name="c", subcore_axis_name="s")
    run = pl.kernel(
        _sc_kernel,
        out_type=jax.ShapeDtypeStruct((NC, N_PAD, D_FEAT), jnp.float32),
        mesh=mesh,
        scratch_types=[
            pltpu.VMEM((NBB, B), jnp.int32),
            pltpu.VMEM((NBB, B), jnp.int32),
            pltpu.VMEM((NBB, B), jnp.float32),
            pltpu.VMEM((B, D_FEAT), jnp.float32),
            pltpu.VMEM((B, D_FEAT), jnp.float32),
            pltpu.VMEM((B, D_FEAT), jnp.float32),
            pltpu.VMEM_SHARED((N_PAD, D_FEAT), jnp.float32),
            pltpu.SemaphoreType.DMA,
            pltpu.SemaphoreType.DMA,
            pltpu.SemaphoreType.DMA,
            pltpu.SemaphoreType.DMA,
            pltpu.SemaphoreType.DMA,
            pltpu.SemaphoreType.DMA,
        ],
    )
    out = run(x, src, dst, t)
    return jnp.swapaxes(out[:, :N_NODES, :], 0, 1)


# speedup vs baseline: 9.0224x; 1.0076x over previous
import functools

import jax
import jax.numpy as jnp
from jax import lax
from jax.experimental import pallas as pl
from jax.experimental.pallas import tpu as pltpu
from jax.experimental.pallas import tpu_sc as plsc

N_NODES = 10000
N_EDGES = 320000
D_FEAT = 128

NC = 2
NS = 16
B = 112
CHUNK = 2 * N_EDGES // (NC * NS)
NBB = 12
NB = -(-CHUNK // (B * NBB)) * NBB
NG = NB // NBB
PAD_CHUNK = NB * B
N_PAD = 10240
ROWS_PER_TILE = N_PAD // NS
RQ = 80


def _sc_kernel(x_hbm, src_hbm, dst_hbm, t_hbm, out_hbm,
               src_v, dst_v, t_v, rows0, rows1, rows2, acc,
               gsem0, gsem1, gsem2, hsem0, hsem1, hsem2,
               ssem0, ssem1, ssem2):
    c = lax.axis_index("c")
    s = lax.axis_index("s")
    w = c * NS + s
    row0 = s * ROWS_PER_TILE
    bufs = ((rows0, gsem0, hsem0, ssem0), (rows1, gsem1, hsem1, ssem1),
            (rows2, gsem2, hsem2, ssem2))
    H = B // 2

    def gather_start(j, rows, semA, semB):
        pltpu.async_copy(x_hbm.at[src_v.at[j, pl.ds(0, H)]],
                         rows.at[pl.ds(0, H)], semA)
        pltpu.async_copy(x_hbm.at[src_v.at[j, pl.ds(H, H)]],
                         rows.at[pl.ds(H, H)], semB)

    def gather_wait(j, rows, semA, semB):
        pltpu.make_async_copy(x_hbm.at[src_v.at[j, pl.ds(0, H)]],
                              rows.at[pl.ds(0, H)], semA).wait()
        pltpu.make_async_copy(x_hbm.at[src_v.at[j, pl.ds(H, H)]],
                              rows.at[pl.ds(H, H)], semB).wait()

    zero16 = jnp.zeros((16,), jnp.float32)

    def zrow(r, carry):
        for k in range(D_FEAT // 16):
            rows0[r, pl.ds(k * 16, 16)] = zero16
        return carry

    lax.fori_loop(0, RQ, zrow, 0)
    for q in range(ROWS_PER_TILE // RQ):
        pltpu.sync_copy(rows0.at[pl.ds(0, RQ)],
                        acc.at[pl.ds(row0 + q * RQ, RQ)])
    plsc.subcore_barrier()

    def scale_rows(rows, j):
        def block(bb, inner):
            t16 = t_v[j, pl.ds(bb * 16, 16)]
            for l in range(16):
                tl = t16[l]
                e = bb * 16 + l
                for k in range(D_FEAT // 16):
                    sl = pl.ds(k * 16, 16)
                    rows[e, sl] = rows[e, sl] * tl
            return inner

        lax.fori_loop(0, B // 16, block, 0)

    def group(g, gcarry):
        @pl.when(g > 0)
        def _():
            pltpu.make_async_copy(rows1, acc.at[dst_v.at[0]], ssem1).wait()
            pltpu.make_async_copy(rows2, acc.at[dst_v.at[0]], ssem2).wait()

        pltpu.sync_copy(src_hbm.at[w, g], src_v)
        pltpu.sync_copy(dst_hbm.at[w, g], dst_v)
        pltpu.sync_copy(t_hbm.at[w, g], t_v)

        gather_start(0, rows0, gsem0, hsem0)

        def triple(p, carry):
            for b in range(3):
                jj = p * 3 + b
                rows, gsem, hsem, _ssem = bufs[b]
                nrows, ngsem, nhsem, nssem = bufs[(b + 1) % 3]

                @pl.when(jj >= 2)
                def _():
                    pltpu.make_async_copy(
                        nrows, acc.at[dst_v.at[0]], nssem).wait()

                @pl.when(jj + 1 < NBB)
                def _():
                    gather_start(jj + 1, nrows, ngsem, nhsem)

                gather_wait(jj, rows, gsem, hsem)
                scale_rows(rows, jj)
                pltpu.async_copy(rows, acc.at[dst_v.at[jj]], _ssem, add=True)
            return carry

        lax.fori_loop(0, NBB // 3, triple, 0)
        return gcarry

    lax.fori_loop(0, NG, group, 0)

    pltpu.make_async_copy(rows1, acc.at[dst_v.at[0]], ssem1).wait()
    pltpu.make_async_copy(rows2, acc.at[dst_v.at[0]], ssem2).wait()
    plsc.subcore_barrier()

    for q in range(ROWS_PER_TILE // RQ):
        r = row0 + q * RQ
        pltpu.sync_copy(acc.at[pl.ds(r, RQ)], rows0.at[pl.ds(0, RQ)])
        pltpu.sync_copy(rows0.at[pl.ds(0, RQ)], out_hbm.at[c, pl.ds(r, RQ)])


@jax.jit
def kernel(x0, x1, edge_index0, edge_index1, t0, t1):
    x = jnp.concatenate([x0, x1], axis=0)
    src = jnp.concatenate([edge_index0[0].astype(jnp.int32),
                           edge_index1[0].astype(jnp.int32) + N_NODES])
    dst = jnp.concatenate([edge_index0[1].astype(jnp.int32),
                           edge_index1[1].astype(jnp.int32)])
    t = jnp.concatenate([t0[:, 0], t1[:, 0]])

    pad = NS * PAD_CHUNK - N_EDGES
    pad_src = (jnp.arange(pad, dtype=jnp.int32) * 37) % N_NODES
    pad_dst = (jnp.arange(pad, dtype=jnp.int32) * 53) % N_NODES
    pad_t = jnp.zeros((pad,), jnp.float32)
    src = jnp.concatenate([src[:N_EDGES], pad_src,
                           src[N_EDGES:], pad_src + N_NODES])
    dst = jnp.concatenate([dst[:N_EDGES], pad_dst,
                           dst[N_EDGES:], pad_dst])
    t = jnp.concatenate([t[:N_EDGES], pad_t, t[N_EDGES:], pad_t])

    src = src.reshape(NC * NS, NG, NBB, B)
    dst = dst.reshape(NC * NS, NG, NBB, B)
    t = t.reshape(NC * NS, NG, NBB, B)

    mesh = plsc.VectorSubcoreMesh(core_axis_name="c", subcore_axis_name="s")
    run = pl.kernel(
        _sc_kernel,
        out_type=jax.ShapeDtypeStruct((NC, N_PAD, D_FEAT), jnp.float32),
        mesh=mesh,
        scratch_types=[
            pltpu.VMEM((NBB, B), jnp.int32),
            pltpu.VMEM((NBB, B), jnp.int32),
            pltpu.VMEM((NBB, B), jnp.float32),
            pltpu.VMEM((B, D_FEAT), jnp.float32),
            pltpu.VMEM((B, D_FEAT), jnp.float32),
            pltpu.VMEM((B, D_FEAT), jnp.float32),
            pltpu.VMEM_SHARED((N_PAD, D_FEAT), jnp.float32),
            pltpu.SemaphoreType.DMA,
            pltpu.SemaphoreType.DMA,
            pltpu.SemaphoreType.DMA,
            pltpu.SemaphoreType.DMA,
            pltpu.SemaphoreType.DMA,
            pltpu.SemaphoreType.DMA,
            pltpu.SemaphoreType.DMA,
            pltpu.SemaphoreType.DMA,
            pltpu.SemaphoreType.DMA,
        ],
    )
    out = run(x, src, dst, t)
    return jnp.swapaxes(out[:, :N_NODES, :], 0, 1)
